# baseline (speedup 1.0000x reference)
import functools

import jax
import jax.numpy as jnp
from jax import lax
from jax.experimental import pallas as pl
from jax.experimental.pallas import tpu as pltpu
from jax.experimental.pallas import tpu_sc as plsc

MAXLEN_ = 200
EMBED_ = 64
BATCH_ = 4096
LANES_ = 16
SPLIT_ = 96
REST_ = MAXLEN_ - SPLIT_
NB_ = 4


def kernel(x, token_table, pos_table):
    info = plsc.get_sparse_core_info()
    nc, ns = info.num_cores, info.num_subcores
    nw = nc * ns
    rows_per_w = BATCH_ // nw
    nch = rows_per_w

    mesh = plsc.VectorSubcoreMesh(core_axis_name="c", subcore_axis_name="s")

    scratch = (
        [pltpu.VMEM((SPLIT_,), jnp.int32) for _ in range(NB_)]
        + [pltpu.VMEM((REST_,), jnp.int32) for _ in range(NB_)]
        + [pltpu.VMEM((MAXLEN_, EMBED_), jnp.float32) for _ in range(NB_)]
        + [pltpu.VMEM((MAXLEN_, EMBED_), jnp.float32)]
        + [pltpu.SemaphoreType.DMA for _ in range(3 * NB_)]
    )

    @functools.partial(
        pl.kernel,
        mesh=mesh,
        compiler_params=pltpu.CompilerParams(use_tc_tiling_on_sc=False),
        out_type=jax.ShapeDtypeStruct((BATCH_, MAXLEN_, 128), jnp.float32),
        scratch_types=scratch,
    )
    def emb_kernel(x_hbm, tt_hbm, pt_hbm, out_hbm, *refs):
        ia = refs[0:NB_]
        ib = refs[NB_ : 2 * NB_]
        tok = refs[2 * NB_ : 3 * NB_]
        pos = refs[3 * NB_]
        isem = refs[3 * NB_ + 1 : 3 * NB_ + 1 + NB_]
        gsem = refs[3 * NB_ + 1 + NB_ : 3 * NB_ + 1 + 2 * NB_]
        osem = refs[3 * NB_ + 1 + 2 * NB_ : 3 * NB_ + 1 + 3 * NB_]

        wid = lax.axis_index("s") * nc + lax.axis_index("c")
        base = wid * rows_per_w

        def issue_idx(c, b):
            off = (base + c) * MAXLEN_
            pltpu.async_copy(x_hbm.at[pl.ds(off, SPLIT_)], ia[b], isem[b])
            pltpu.async_copy(x_hbm.at[pl.ds(off + SPLIT_, REST_)], ib[b], isem[b])

        def wait_idx(b):
            pltpu.make_async_copy(x_hbm.at[pl.ds(0, SPLIT_)], ia[b], isem[b]).wait()
            pltpu.make_async_copy(x_hbm.at[pl.ds(0, REST_)], ib[b], isem[b]).wait()

        def issue_gather(b):
            pltpu.async_copy(tt_hbm.at[ia[b]], tok[b].at[pl.ds(0, SPLIT_)], gsem[b])
            pltpu.async_copy(
                tt_hbm.at[ib[b]], tok[b].at[pl.ds(SPLIT_, REST_)], gsem[b]
            )

        def wait_gather(b):
            pltpu.make_async_copy(
                tt_hbm.at[ia[b]], tok[b].at[pl.ds(0, SPLIT_)], gsem[b]
            ).wait()
            pltpu.make_async_copy(
                tt_hbm.at[ib[b]], tok[b].at[pl.ds(SPLIT_, REST_)], gsem[b]
            ).wait()

        def issue_out(c, b):
            pltpu.async_copy(
                tok[b], out_hbm.at[base + c, :, pl.ds(0, EMBED_)], osem[b]
            )

        def wait_out(b):
            pltpu.make_async_copy(
                tok[b], out_hbm.at[base, :, pl.ds(0, EMBED_)], osem[b]
            ).wait()

        def add(b):
            tok_b = tok[b]

            def add_rows(i2, carry):
                for r in range(2):
                    i = i2 * 2 + r
                    for j in range(EMBED_ // LANES_):
                        sl = pl.ds(j * LANES_, LANES_)
                        tok_b[i, sl] = tok_b[i, sl] + pos[i, sl]
                return carry

            lax.fori_loop(0, MAXLEN_ // 2, add_rows, 0)

        def step(c, b, *, with_out_wait):
            issue_idx(c + 2, (b + 2) % NB_)
            wait_idx((b + 1) % NB_)
            if with_out_wait:
                wait_out((b + 1) % NB_)
            issue_gather((b + 1) % NB_)
            wait_gather(b)
            add(b)
            issue_out(c, b)

        pltpu.sync_copy(pt_hbm, pos)

        issue_idx(0, 0)
        issue_idx(1, 1)
        wait_idx(0)
        issue_gather(0)
        step(0, 0, with_out_wait=False)
        step(1, 1, with_out_wait=False)
        step(2, 2, with_out_wait=False)
        step(3, 3, with_out_wait=True)

        def group(g, carry):
            c0 = g * NB_
            for b in range(NB_):
                step(c0 + b, b, with_out_wait=True)
            return carry

        lax.fori_loop(1, nch // NB_ - 1, group, 0)

        c0 = nch - NB_
        issue_idx(c0 + 2, 2)
        wait_idx(1)
        wait_out(1)
        issue_gather(1)
        wait_gather(0)
        add(0)
        issue_out(c0, 0)
        issue_idx(c0 + 3, 3)
        wait_idx(2)
        wait_out(2)
        issue_gather(2)
        wait_gather(1)
        add(1)
        issue_out(c0 + 1, 1)
        wait_idx(3)
        wait_out(3)
        issue_gather(3)
        wait_gather(2)
        add(2)
        issue_out(c0 + 2, 2)
        wait_gather(3)
        add(3)
        issue_out(c0 + 3, 3)
        for b in range(NB_):
            wait_out(b)

    out128 = emb_kernel(x.astype(jnp.int32).reshape(-1), token_table, pos_table)
    return out128[:, :, :EMBED_]
